# baseline (speedup 1.0000x reference)
import functools

import jax
import jax.numpy as jnp
from jax import lax
from jax.experimental import pallas as pl
from jax.experimental.pallas import tpu as pltpu
from jax.experimental.pallas import tpu_sc as plsc

_VOCAB = 100000
_EMBD = 128
_CTX = 10
_HID = 512
_BN = 4096
_NB = (_VOCAB + _BN - 1) // _BN


def _sc_gather(idx, emb):
    n = idx.shape[0]
    mesh = plsc.VectorSubcoreMesh(core_axis_name="c", subcore_axis_name="s")

    @functools.partial(
        pl.kernel,
        out_type=jax.ShapeDtypeStruct((n, _EMBD), jnp.float32),
        mesh=mesh,
        scratch_types=[
            pltpu.VMEM((n,), jnp.int32),
            pltpu.VMEM((n, _EMBD), jnp.float32),
            pltpu.SemaphoreType.DMA,
        ],
    )
    def k(idx_hbm, emb_hbm, out_hbm, idx_v, rows_v, sem):
        c = lax.axis_index("c")
        s = lax.axis_index("s")

        @pl.when(jnp.logical_and(c == 0, s == 0))
        def _():
            pltpu.sync_copy(idx_hbm, idx_v)
            pltpu.async_copy(emb_hbm.at[idx_v], rows_v, sem).wait()
            pltpu.sync_copy(rows_v, out_hbm)

    return k(idx, emb)


def _mlp_body(e_ref, w1_ref, b1_ref, w2_ref, b2_ref, out_ref, hid_s, out_s, sm):
    p = pl.program_id(0)
    i = pl.program_id(1)

    @pl.when(jnp.logical_and(p == 0, i == 0))
    def _():
        h = jnp.dot(e_ref[...], w1_ref[...], preferred_element_type=jnp.float32)
        hid_s[...] = jnp.maximum(h + b1_ref[...], 0.0)
        sm[0] = -jnp.inf
        sm[1] = 0.0

    @pl.when(p == 0)
    def _():
        blk = jnp.dot(hid_s[...], w2_ref[...],
                      preferred_element_type=jnp.float32) + b2_ref[...]
        col = i * _BN + lax.broadcasted_iota(jnp.int32, (1, _BN), 1)
        valid = col < _VOCAB
        blkm = jnp.where(valid, blk, -jnp.inf)
        out_s[:, pl.ds(i * _BN, _BN)] = blk
        m0 = sm[0]
        m1 = jnp.maximum(m0, jnp.max(blkm))
        s1 = sm[1] * jnp.exp(m0 - m1) + jnp.sum(
            jnp.where(valid, jnp.exp(blkm - m1), 0.0))
        sm[0] = m1
        sm[1] = s1

        @pl.when(i == _NB - 1)
        def _():
            sm[0] = m1 + jnp.log(s1)

    @pl.when(p == 1)
    def _():
        out_ref[...] = out_s[:, pl.ds(i * _BN, _BN)] - sm[0]


def _tc_mlp(embedded, W1, b1_row, W2, b2_row):
    return pl.pallas_call(
        _mlp_body,
        grid=(2, _NB),
        in_specs=[
            pl.BlockSpec((1, 2 * _CTX * _EMBD), lambda p, i: (0, 0)),
            pl.BlockSpec((2 * _CTX * _EMBD, _HID), lambda p, i: (0, 0)),
            pl.BlockSpec((1, _HID), lambda p, i: (0, 0)),
            pl.BlockSpec((_HID, _BN),
                         lambda p, i: (0, i * (1 - p) + (_NB - 1) * p)),
            pl.BlockSpec((1, _BN),
                         lambda p, i: (0, i * (1 - p) + (_NB - 1) * p)),
        ],
        out_specs=pl.BlockSpec((1, _BN), lambda p, i: (0, i * p)),
        out_shape=jax.ShapeDtypeStruct((1, _VOCAB), jnp.float32),
        scratch_shapes=[
            pltpu.VMEM((1, _HID), jnp.float32),
            pltpu.VMEM((1, _NB * _BN), jnp.float32),
            pltpu.SMEM((2,), jnp.float32),
        ],
        compiler_params=pltpu.CompilerParams(
            dimension_semantics=("arbitrary", "arbitrary"),
        ),
    )(embedded, W1, b1_row, W2, b2_row)


def kernel(inputs, emb, W1, b1, W2, b2):
    embedded = jnp.take(emb, inputs, axis=0).reshape(1, 2 * _CTX * _EMBD)
    return _tc_mlp(embedded, W1, b1.reshape(1, _HID), W2,
                   b2.reshape(1, _VOCAB))


# speedup vs baseline: 1.0104x; 1.0104x over previous
import functools

import jax
import jax.numpy as jnp
from jax import lax
from jax.experimental import pallas as pl
from jax.experimental.pallas import tpu as pltpu
from jax.experimental.pallas import tpu_sc as plsc

_VOCAB = 100000
_EMBD = 128
_CTX = 10
_HID = 512
_BN = 1280
_NB = (_VOCAB + _BN - 1) // _BN
_K = 4
_NS = (_NB + _K - 1) // _K


def _sc_gather(idx, emb):
    n = idx.shape[0]
    mesh = plsc.VectorSubcoreMesh(core_axis_name="c", subcore_axis_name="s")

    @functools.partial(
        pl.kernel,
        out_type=jax.ShapeDtypeStruct((n, _EMBD), jnp.float32),
        mesh=mesh,
        scratch_types=[
            pltpu.VMEM((n,), jnp.int32),
            pltpu.VMEM((n, _EMBD), jnp.float32),
            pltpu.SemaphoreType.DMA,
        ],
    )
    def k(idx_hbm, emb_hbm, out_hbm, idx_v, rows_v, sem):
        c = lax.axis_index("c")
        s = lax.axis_index("s")

        @pl.when(jnp.logical_and(c == 0, s == 0))
        def _():
            pltpu.sync_copy(idx_hbm, idx_v)
            pltpu.async_copy(emb_hbm.at[idx_v], rows_v, sem).wait()
            pltpu.sync_copy(rows_v, out_hbm)

    return k(idx, emb)


def _w2_index_map(j):
    def index_map(p, i):
        q = jnp.minimum(_K * i + j, _NB - 1)
        return (0, jnp.where(p == 0, q, _NB - 1))
    return index_map


def _mlp_body(e_ref, w1_ref, b1_ref, *rest):
    w2_refs = rest[:_K]
    b2_refs = rest[_K:2 * _K]
    out_ref = rest[2 * _K]
    hid_s, out_s, sm = rest[2 * _K + 1:]

    p = pl.program_id(0)
    i = pl.program_id(1)

    @pl.when(jnp.logical_and(p == 0, i == 0))
    def _():
        h = jnp.dot(e_ref[...], w1_ref[...], preferred_element_type=jnp.float32)
        hid_s[...] = jnp.maximum(h + b1_ref[...], 0.0)
        sm[0] = -jnp.inf
        sm[1] = 0.0

    @pl.when(p == 0)
    def _():
        m0 = sm[0]
        s0 = sm[1]
        m1 = m0
        blks = []
        for j in range(_K):
            blk = jnp.dot(hid_s[...], w2_refs[j][...],
                          preferred_element_type=jnp.float32) + b2_refs[j][...]
            base = (_K * i + j) * _BN
            col = base + lax.broadcasted_iota(jnp.int32, (1, _BN), 1)
            valid = col < _VOCAB
            blkm = jnp.where(valid, blk, -jnp.inf)
            out_s[:, pl.ds(base, _BN)] = blk
            blks.append(blkm)
            m1 = jnp.maximum(m1, jnp.max(blkm))
        s1 = s0 * jnp.exp(m0 - m1)
        for j in range(_K):
            s1 = s1 + jnp.sum(jnp.exp(blks[j] - m1))
        sm[0] = m1
        sm[1] = s1

        @pl.when(i == _NS - 1)
        def _():
            sm[0] = m1 + jnp.log(s1)

    @pl.when(p == 1)
    def _():
        out_ref[...] = out_s[:, pl.ds(i * _K * _BN, _K * _BN)] - sm[0]


def _tc_mlp(embedded, W1, b1_row, W2, b2_row):
    in_specs = [
        pl.BlockSpec((1, 2 * _CTX * _EMBD), lambda p, i: (0, 0)),
        pl.BlockSpec((2 * _CTX * _EMBD, _HID), lambda p, i: (0, 0)),
        pl.BlockSpec((1, _HID), lambda p, i: (0, 0)),
    ]
    in_specs += [pl.BlockSpec((_HID, _BN), _w2_index_map(j)) for j in range(_K)]
    in_specs += [pl.BlockSpec((1, _BN), _w2_index_map(j)) for j in range(_K)]
    operands = [embedded, W1, b1_row] + [W2] * _K + [b2_row] * _K
    return pl.pallas_call(
        _mlp_body,
        grid=(2, _NS),
        in_specs=in_specs,
        out_specs=pl.BlockSpec((1, _K * _BN), lambda p, i: (0, i * p)),
        out_shape=jax.ShapeDtypeStruct((1, _VOCAB), jnp.float32),
        scratch_shapes=[
            pltpu.VMEM((1, _HID), jnp.float32),
            pltpu.VMEM((1, _NS * _K * _BN), jnp.float32),
            pltpu.SMEM((2,), jnp.float32),
        ],
        compiler_params=pltpu.CompilerParams(
            dimension_semantics=("arbitrary", "arbitrary"),
        ),
    )(*operands)


def kernel(inputs, emb, W1, b1, W2, b2):
    embedded = _sc_gather(inputs, emb).reshape(1, 2 * _CTX * _EMBD)
    return _tc_mlp(embedded, W1, b1.reshape(1, _HID), W2,
                   b2.reshape(1, _VOCAB))


# speedup vs baseline: 1.0230x; 1.0125x over previous
import functools

import jax
import jax.numpy as jnp
from jax import lax
from jax.experimental import pallas as pl
from jax.experimental.pallas import tpu as pltpu
from jax.experimental.pallas import tpu_sc as plsc

_VOCAB = 100000
_EMBD = 128
_CTX = 10
_HID = 512
_BN = 1280
_NC = _VOCAB // _BN
_TAIL = _VOCAB - _NC * _BN
_NBUF = 6
_NSTEP = _NC // _NBUF


def _sc_gather(idx, emb):
    n = idx.shape[0]
    mesh = plsc.VectorSubcoreMesh(core_axis_name="c", subcore_axis_name="s")

    @functools.partial(
        pl.kernel,
        out_type=jax.ShapeDtypeStruct((n, _EMBD), jnp.float32),
        mesh=mesh,
        scratch_types=[
            pltpu.VMEM((n,), jnp.int32),
            pltpu.VMEM((n, _EMBD), jnp.float32),
            pltpu.SemaphoreType.DMA,
        ],
    )
    def k(idx_hbm, emb_hbm, out_hbm, idx_v, rows_v, sem):
        c = lax.axis_index("c")
        s = lax.axis_index("s")

        @pl.when(jnp.logical_and(c == 0, s == 0))
        def _():
            pltpu.sync_copy(idx_hbm, idx_v)
            pltpu.async_copy(emb_hbm.at[idx_v], rows_v, sem).wait()
            pltpu.sync_copy(rows_v, out_hbm)

    return k(idx, emb)


def _chunk_copy(w2_any, buf_s, sems, c, j):
    return pltpu.make_async_copy(
        w2_any.at[:, pl.ds(c * _BN, _BN)],
        buf_s.at[j],
        sems.at[j],
    )


def _mega_body(e_ref, w1_ref, b1_ref, b2_ref, w2_any, out_ref,
               hid_s, out_s, buf_s, tail_s, sems, tail_sem):
    for j in range(_NBUF):
        _chunk_copy(w2_any, buf_s, sems, j, j).start()
    pltpu.make_async_copy(
        w2_any.at[:, pl.ds(_NC * _BN, _TAIL)], tail_s, tail_sem).start()

    h = jnp.dot(e_ref[...], w1_ref[...], preferred_element_type=jnp.float32)
    hid_s[...] = jnp.maximum(h + b1_ref[...], 0.0)

    def step(s, carry):
        m0, s0 = carry
        for j in range(_NBUF):
            c = _NBUF * s + j
            _chunk_copy(w2_any, buf_s, sems, c, j).wait()
            blk = jnp.dot(hid_s[...], buf_s[j],
                          preferred_element_type=jnp.float32)
            blk = blk + b2_ref[:, pl.ds(c * _BN, _BN)]
            out_s[:, pl.ds(c * _BN, _BN)] = blk
            m1 = jnp.maximum(m0, jnp.max(blk))
            s0 = s0 * jnp.exp(m0 - m1) + jnp.sum(jnp.exp(blk - m1))
            m0 = m1

            @pl.when(c + _NBUF < _NC)
            def _():
                _chunk_copy(w2_any, buf_s, sems, c + _NBUF, j).start()
        return m0, s0

    m0, s0 = lax.fori_loop(
        0, _NSTEP, step, (jnp.float32(-jnp.inf), jnp.float32(0.0)))

    pltpu.make_async_copy(
        w2_any.at[:, pl.ds(_NC * _BN, _TAIL)], tail_s, tail_sem).wait()
    blk = jnp.dot(hid_s[...], tail_s[...], preferred_element_type=jnp.float32)
    blk = blk + b2_ref[:, pl.ds(_NC * _BN, _TAIL)]
    out_s[:, pl.ds(_NC * _BN, _TAIL)] = blk
    m1 = jnp.maximum(m0, jnp.max(blk))
    s1 = s0 * jnp.exp(m0 - m1) + jnp.sum(jnp.exp(blk - m1))

    lse = m1 + jnp.log(s1)
    out_ref[...] = out_s[...] - lse


def _tc_mlp(embedded, W1, b1_row, W2, b2_row):
    return pl.pallas_call(
        _mega_body,
        in_specs=[
            pl.BlockSpec(memory_space=pltpu.VMEM),
            pl.BlockSpec(memory_space=pltpu.VMEM),
            pl.BlockSpec(memory_space=pltpu.VMEM),
            pl.BlockSpec(memory_space=pltpu.VMEM),
            pl.BlockSpec(memory_space=pl.ANY),
        ],
        out_specs=pl.BlockSpec(memory_space=pltpu.VMEM),
        out_shape=jax.ShapeDtypeStruct((1, _VOCAB), jnp.float32),
        scratch_shapes=[
            pltpu.VMEM((1, _HID), jnp.float32),
            pltpu.VMEM((1, _VOCAB), jnp.float32),
            pltpu.VMEM((_NBUF, _HID, _BN), jnp.float32),
            pltpu.VMEM((_HID, _TAIL), jnp.float32),
            pltpu.SemaphoreType.DMA((_NBUF,)),
            pltpu.SemaphoreType.DMA,
        ],
    )(embedded, W1, b1_row, b2_row, W2)


def kernel(inputs, emb, W1, b1, W2, b2):
    embedded = _sc_gather(inputs, emb).reshape(1, 2 * _CTX * _EMBD)
    return _tc_mlp(embedded, W1, b1.reshape(1, _HID), W2,
                   b2.reshape(1, _VOCAB))


# speedup vs baseline: 2.9891x; 2.9220x over previous
import functools

import jax
import jax.numpy as jnp
from jax import lax
from jax.experimental import pallas as pl
from jax.experimental.pallas import tpu as pltpu
from jax.experimental.pallas import tpu_sc as plsc

_VOCAB = 100000
_EMBD = 128
_CTX = 10
_HID = 512
_BN = 1280
_NC = _VOCAB // _BN
_TAIL = _VOCAB - _NC * _BN
_NBUF = 6
_NSTEP = _NC // _NBUF


def _sc_gather(idx, emb):
    n = idx.shape[0]
    mesh = plsc.VectorSubcoreMesh(core_axis_name="c", subcore_axis_name="s")

    @functools.partial(
        pl.kernel,
        out_type=jax.ShapeDtypeStruct((n, _EMBD), jnp.float32),
        mesh=mesh,
        scratch_types=[
            pltpu.VMEM((n,), jnp.int32),
            pltpu.VMEM((n, _EMBD), jnp.float32),
            pltpu.SemaphoreType.DMA,
        ],
    )
    def k(idx_hbm, emb_hbm, out_hbm, idx_v, rows_v, sem):
        c = lax.axis_index("c")
        s = lax.axis_index("s")

        @pl.when(jnp.logical_and(c == 0, s == 0))
        def _():
            pltpu.sync_copy(idx_hbm, idx_v)
            pltpu.async_copy(emb_hbm.at[idx_v], rows_v, sem).wait()
            pltpu.sync_copy(rows_v, out_hbm)

    return k(idx, emb)


def _chunk_copy(w2t_any, buf_s, sems, c, j):
    return pltpu.make_async_copy(
        w2t_any.at[pl.ds(c * _BN, _BN), :],
        buf_s.at[j],
        sems.at[j],
    )


def _dotT(hid, chunk):
    return lax.dot_general(hid, chunk, (((1,), (1,)), ((), ())),
                           preferred_element_type=jnp.float32)


def _mega_body(e_ref, w1_ref, b1_ref, b2_ref, w2t_any, out_ref,
               hid_s, out_s, buf_s, tail_s, sems, tail_sem):
    for j in range(_NBUF):
        _chunk_copy(w2t_any, buf_s, sems, j, j).start()
    pltpu.make_async_copy(
        w2t_any.at[pl.ds(_NC * _BN, _TAIL), :], tail_s, tail_sem).start()

    h = jnp.dot(e_ref[...], w1_ref[...], preferred_element_type=jnp.float32)
    hid_s[...] = jnp.maximum(h + b1_ref[...], 0.0)

    def step(s, carry):
        m0, s0 = carry
        for j in range(_NBUF):
            c = _NBUF * s + j
            _chunk_copy(w2t_any, buf_s, sems, c, j).wait()
            blk = _dotT(hid_s[...], buf_s[j])
            blk = blk + b2_ref[:, pl.ds(c * _BN, _BN)]
            out_s[:, pl.ds(c * _BN, _BN)] = blk
            m1 = jnp.maximum(m0, jnp.max(blk))
            s0 = s0 * jnp.exp(m0 - m1) + jnp.sum(jnp.exp(blk - m1))
            m0 = m1

            @pl.when(c + _NBUF < _NC)
            def _():
                _chunk_copy(w2t_any, buf_s, sems, c + _NBUF, j).start()
        return m0, s0

    m0, s0 = lax.fori_loop(
        0, _NSTEP, step, (jnp.float32(-jnp.inf), jnp.float32(0.0)))

    pltpu.make_async_copy(
        w2t_any.at[pl.ds(_NC * _BN, _TAIL), :], tail_s, tail_sem).wait()
    blk = _dotT(hid_s[...], tail_s[...])
    blk = blk + b2_ref[:, pl.ds(_NC * _BN, _TAIL)]
    out_s[:, pl.ds(_NC * _BN, _TAIL)] = blk
    m1 = jnp.maximum(m0, jnp.max(blk))
    s1 = s0 * jnp.exp(m0 - m1) + jnp.sum(jnp.exp(blk - m1))

    lse = m1 + jnp.log(s1)
    out_ref[...] = out_s[...] - lse


def _tc_mlp(embedded, W1, b1_row, W2T, b2_row):
    return pl.pallas_call(
        _mega_body,
        in_specs=[
            pl.BlockSpec(memory_space=pltpu.VMEM),
            pl.BlockSpec(memory_space=pltpu.VMEM),
            pl.BlockSpec(memory_space=pltpu.VMEM),
            pl.BlockSpec(memory_space=pltpu.VMEM),
            pl.BlockSpec(memory_space=pl.ANY),
        ],
        out_specs=pl.BlockSpec(memory_space=pltpu.VMEM),
        out_shape=jax.ShapeDtypeStruct((1, _VOCAB), jnp.float32),
        scratch_shapes=[
            pltpu.VMEM((1, _HID), jnp.float32),
            pltpu.VMEM((1, _VOCAB), jnp.float32),
            pltpu.VMEM((_NBUF, _BN, _HID), jnp.float32),
            pltpu.VMEM((_TAIL, _HID), jnp.float32),
            pltpu.SemaphoreType.DMA((_NBUF,)),
            pltpu.SemaphoreType.DMA,
        ],
    )(embedded, W1, b1_row, b2_row, W2T)


def kernel(inputs, emb, W1, b1, W2, b2):
    embedded = _sc_gather(inputs, emb).reshape(1, 2 * _CTX * _EMBD)
    return _tc_mlp(embedded, W1, b1.reshape(1, _HID), jnp.swapaxes(W2, 0, 1),
                   b2.reshape(1, _VOCAB))
